# Initial kernel scaffold; baseline (speedup 1.0000x reference)
import functools

import jax
import jax.numpy as jnp
from jax import lax
from jax.experimental import pallas as pl
from jax.experimental.pallas import tpu as pltpu
from jax.experimental.pallas import tpu_sc as plsc


def _make_sc_gather(V, D, N):
    info = plsc.get_sparse_core_info()
    NC, NS = info.num_cores, info.num_subcores
    NW = NC * NS
    K = 16
    C = K * 128
    b_per_w = N // NW
    assert N % (NW * C) == 0, (N, NW, C)
    steps = b_per_w // C
    mesh = plsc.VectorSubcoreMesh(core_axis_name="c", subcore_axis_name="s")

    @functools.partial(
        pl.kernel,
        mesh=mesh,
        out_type=jax.ShapeDtypeStruct((N, D), jnp.float32),
        scratch_types=[
            pltpu.VMEM((K, 128), jnp.int32),
            pltpu.VMEM((C, D), jnp.float32),
            pltpu.SemaphoreType.DMA,
        ],
    )
    def gather_kernel(table_hbm, idx_hbm, out_hbm, idx_v, rows_v, sem):
        wid = lax.axis_index("s") * NC + lax.axis_index("c")
        idx_row0 = wid * (b_per_w // 128)

        def step(i, carry):
            r0 = idx_row0 + i * K
            pltpu.sync_copy(idx_hbm.at[pl.ds(r0, K)], idx_v)
            copies = [
                pltpu.async_copy(
                    table_hbm.at[idx_v.at[j]],
                    rows_v.at[pl.ds(j * 128, 128)],
                    sem,
                )
                for j in range(K)
            ]
            for c in copies:
                c.wait()
            pltpu.sync_copy(rows_v, out_hbm.at[pl.ds(r0 * 128, C)])
            return carry

        lax.fori_loop(0, steps, step, 0)

    return gather_kernel


def _make_tc_fused(N, F, ND, NT, DE, DL, H, BLK):
    grid = N // BLK
    assert N % BLK == 0

    def body(dn_ref, dt_ref, loc_ref, ed_ref, et_ref, w_ref, b_ref, out_ref):
        dn = dn_ref[...]
        dt = dt_ref[...]
        loc = loc_ref[...]
        W = w_ref[...]
        d = dt[:, 0:1]
        t = dt[:, 1:2]
        oh_d = (lax.broadcasted_iota(jnp.int32, (BLK, ND), 1) == d).astype(
            jnp.float32)
        oh_t = (lax.broadcasted_iota(jnp.int32, (BLK, NT), 1) == t).astype(
            jnp.float32)
        e0 = jnp.dot(oh_d, ed_ref[...], preferred_element_type=jnp.float32)
        e1 = jnp.dot(oh_t, et_ref[...], preferred_element_type=jnp.float32)
        acc = jnp.dot(dn, W[0:F, :], preferred_element_type=jnp.float32)
        acc = acc + jnp.dot(e0, W[F:F + DE, :],
                            preferred_element_type=jnp.float32)
        acc = acc + jnp.dot(e1, W[F + DE:F + 2 * DE, :],
                            preferred_element_type=jnp.float32)
        acc = acc + jnp.dot(loc, W[F + 2 * DE:, :],
                            preferred_element_type=jnp.float32)
        acc = acc + b_ref[...]
        out_ref[...] = jnp.maximum(acc, 0.0)

    return pl.pallas_call(
        body,
        grid=(grid,),
        in_specs=[
            pl.BlockSpec((BLK, F), lambda i: (i, 0)),
            pl.BlockSpec((BLK, 2), lambda i: (i, 0)),
            pl.BlockSpec((BLK, DL), lambda i: (i, 0)),
            pl.BlockSpec((ND, DE), lambda i: (0, 0)),
            pl.BlockSpec((NT, DE), lambda i: (0, 0)),
            pl.BlockSpec((F + 2 * DE + DL, H), lambda i: (0, 0)),
            pl.BlockSpec((1, H), lambda i: (0, 0)),
        ],
        out_specs=pl.BlockSpec((BLK, H), lambda i: (i, 0)),
        out_shape=jax.ShapeDtypeStruct((N, H), jnp.float32),
    )


def kernel(data_num, data_cat, emb_day, emb_time, emb_loc, W_in, b_in):
    B, T, F = data_num.shape
    N = B * T
    V, DL = emb_loc.shape
    NT, DE = emb_time.shape
    H = W_in.shape[1]

    dn = data_num.reshape(N, F)
    dt = data_cat[:, :, 0:2].reshape(N, 2)
    loc_idx = data_cat[:, :, 2].reshape(N // 128, 128)

    loc_rows = _make_sc_gather(V, DL, N)(emb_loc, loc_idx)

    ND = 8
    ed = jnp.zeros((ND, DE), emb_day.dtype).at[:emb_day.shape[0]].set(emb_day)

    out = _make_tc_fused(N, F, ND, NT, DE, DL, H, BLK=4096)(
        dn, dt, loc_rows, ed, emb_time, W_in, b_in.reshape(1, H))
    return out.reshape(B, T, H)


# speedup vs baseline: 7.1239x; 7.1239x over previous
import functools

import jax
import jax.numpy as jnp
from jax import lax
from jax.experimental import pallas as pl
from jax.experimental.pallas import tpu as pltpu
from jax.experimental.pallas import tpu_sc as plsc


def _make_sc_gather(V, D, N):
    info = plsc.get_sparse_core_info()
    NC, NS = info.num_cores, info.num_subcores
    NW = NC * NS
    K = 16
    C = K * 128
    b_per_w = N // NW
    assert N % (NW * C) == 0, (N, NW, C)
    steps = b_per_w // C
    mesh = plsc.VectorSubcoreMesh(core_axis_name="c", subcore_axis_name="s")

    @functools.partial(
        pl.kernel,
        mesh=mesh,
        out_type=jax.ShapeDtypeStruct((N, D), jnp.float32),
        scratch_types=[
            pltpu.VMEM((K, 128), jnp.int32),
            pltpu.VMEM((C, D), jnp.float32),
            pltpu.SemaphoreType.DMA,
        ],
        compiler_params=pltpu.CompilerParams(use_tc_tiling_on_sc=False),
    )
    def gather_kernel(table_hbm, idx_hbm, out_hbm, idx_v, rows_v, sem):
        wid = lax.axis_index("s") * NC + lax.axis_index("c")
        idx_row0 = wid * (b_per_w // 128)

        def step(i, carry):
            r0 = idx_row0 + i * K
            pltpu.sync_copy(idx_hbm.at[pl.ds(r0, K)], idx_v)
            copies = [
                pltpu.async_copy(
                    table_hbm.at[idx_v.at[j]],
                    rows_v.at[pl.ds(j * 128, 128)],
                    sem,
                )
                for j in range(K)
            ]
            for c in copies:
                c.wait()
            pltpu.sync_copy(rows_v, out_hbm.at[pl.ds(r0 * 128, C)])
            return carry

        lax.fori_loop(0, steps, step, 0)

    return gather_kernel


def _make_tc_fused(N, F, ND, NT, DE, DL, H, BLK):
    grid = N // BLK
    assert N % BLK == 0

    def body(dn_ref, dt_ref, loc_ref, ed_ref, et_ref, w_ref, b_ref, out_ref):
        dn = dn_ref[...]
        dt = dt_ref[...]
        loc = loc_ref[...]
        W = w_ref[...]
        d = dt[:, 0:1]
        t = dt[:, 1:2]
        oh_d = (lax.broadcasted_iota(jnp.int32, (BLK, ND), 1) == d).astype(
            jnp.float32)
        oh_t = (lax.broadcasted_iota(jnp.int32, (BLK, NT), 1) == t).astype(
            jnp.float32)
        e0 = jnp.dot(oh_d, ed_ref[...], preferred_element_type=jnp.float32)
        e1 = jnp.dot(oh_t, et_ref[...], preferred_element_type=jnp.float32)
        acc = jnp.dot(dn, W[0:F, :], preferred_element_type=jnp.float32)
        acc = acc + jnp.dot(e0, W[F:F + DE, :],
                            preferred_element_type=jnp.float32)
        acc = acc + jnp.dot(e1, W[F + DE:F + 2 * DE, :],
                            preferred_element_type=jnp.float32)
        acc = acc + jnp.dot(loc, W[F + 2 * DE:, :],
                            preferred_element_type=jnp.float32)
        acc = acc + b_ref[...]
        out_ref[...] = jnp.maximum(acc, 0.0)

    return pl.pallas_call(
        body,
        grid=(grid,),
        in_specs=[
            pl.BlockSpec((BLK, F), lambda i: (i, 0)),
            pl.BlockSpec((BLK, 2), lambda i: (i, 0)),
            pl.BlockSpec((BLK, DL), lambda i: (i, 0)),
            pl.BlockSpec((ND, DE), lambda i: (0, 0)),
            pl.BlockSpec((NT, DE), lambda i: (0, 0)),
            pl.BlockSpec((F + 2 * DE + DL, H), lambda i: (0, 0)),
            pl.BlockSpec((1, H), lambda i: (0, 0)),
        ],
        out_specs=pl.BlockSpec((BLK, H), lambda i: (i, 0)),
        out_shape=jax.ShapeDtypeStruct((N, H), jnp.float32),
    )


def kernel(data_num, data_cat, emb_day, emb_time, emb_loc, W_in, b_in):
    B, T, F = data_num.shape
    N = B * T
    V, DL = emb_loc.shape
    NT, DE = emb_time.shape
    H = W_in.shape[1]

    dn = data_num.reshape(N, F)
    dt = data_cat[:, :, 0:2].reshape(N, 2)
    loc_idx = data_cat[:, :, 2].reshape(N // 128, 128)

    loc_rows = _make_sc_gather(V, DL, N)(emb_loc, loc_idx)

    ND = 8
    ed = jnp.zeros((ND, DE), emb_day.dtype).at[:emb_day.shape[0]].set(emb_day)

    out = _make_tc_fused(N, F, ND, NT, DE, DL, H, BLK=4096)(
        dn, dt, loc_rows, ed, emb_time, W_in, b_in.reshape(1, H))
    return out.reshape(B, T, H)


# speedup vs baseline: 13.6560x; 1.9169x over previous
import functools

import jax
import jax.numpy as jnp
from jax import lax
from jax.experimental import pallas as pl
from jax.experimental.pallas import tpu as pltpu
from jax.experimental.pallas import tpu_sc as plsc


def _make_sc_gather(V, D, N):
    info = plsc.get_sparse_core_info()
    NC, NS = info.num_cores, info.num_subcores
    NW = NC * NS
    K = 16
    C = K * 128
    b_per_w = N // NW
    assert N % (NW * C) == 0, (N, NW, C)
    steps = b_per_w // C
    mesh = plsc.VectorSubcoreMesh(core_axis_name="c", subcore_axis_name="s")

    @functools.partial(
        pl.kernel,
        mesh=mesh,
        out_type=jax.ShapeDtypeStruct((N, D), jnp.float32),
        scratch_types=[
            pltpu.VMEM((K, 128), jnp.int32),
            pltpu.VMEM((C, D), jnp.float32),
            pltpu.SemaphoreType.DMA,
        ],
        compiler_params=pltpu.CompilerParams(use_tc_tiling_on_sc=False),
    )
    def gather_kernel(table_hbm, idx_hbm, out_hbm, idx_v, rows_v, sem):
        wid = lax.axis_index("s") * NC + lax.axis_index("c")
        idx_row0 = wid * (b_per_w // 128)

        def step(i, carry):
            r0 = idx_row0 + i * K
            pltpu.sync_copy(idx_hbm.at[pl.ds(r0, K)], idx_v)
            copies = [
                pltpu.async_copy(
                    table_hbm.at[idx_v.at[j]],
                    rows_v.at[pl.ds(j * 128, 128)],
                    sem,
                )
                for j in range(K)
            ]
            for c in copies:
                c.wait()
            pltpu.sync_copy(rows_v, out_hbm.at[pl.ds(r0 * 128, C)])
            return carry

        lax.fori_loop(0, steps, step, 0)

    return gather_kernel


def _make_tc_fused(B, T, F, NT, DE, DL, H, BLK):
    nb = B // BLK
    assert B % BLK == 0
    ND = 8

    def body(dn_ref, cat_ref, loc_ref, ed_ref, et_ref, w_ref, b_ref,
             out_ref):
        dn = jnp.squeeze(dn_ref[...], axis=0)
        cat = jnp.squeeze(cat_ref[...], axis=0)
        loc = loc_ref[...]
        W = w_ref[...]
        d = cat[0:1, :]
        t = cat[1:2, :]
        oh_d = (lax.broadcasted_iota(jnp.int32, (ND, BLK), 0) == d).astype(
            jnp.float32)
        oh_t = (lax.broadcasted_iota(jnp.int32, (NT, BLK), 0) == t).astype(
            jnp.float32)
        pd = jnp.dot(ed_ref[...], W[F:F + DE, :],
                     preferred_element_type=jnp.float32)
        pt = jnp.dot(et_ref[...], W[F + DE:F + 2 * DE, :],
                     preferred_element_type=jnp.float32)
        cdims = (((0,), (0,)), ((), ()))
        acc = lax.dot_general(dn, W[0:F, :], cdims,
                              preferred_element_type=jnp.float32)
        acc = acc + lax.dot_general(oh_d, pd, cdims,
                                    preferred_element_type=jnp.float32)
        acc = acc + lax.dot_general(oh_t, pt, cdims,
                                    preferred_element_type=jnp.float32)
        acc = acc + jnp.dot(loc, W[F + 2 * DE:, :],
                            preferred_element_type=jnp.float32)
        acc = acc + b_ref[...]
        out_ref[...] = jnp.maximum(acc, 0.0)[None]

    return pl.pallas_call(
        body,
        grid=(T, nb),
        in_specs=[
            pl.BlockSpec((1, F, BLK), lambda i, j: (i, 0, j)),
            pl.BlockSpec((1, 3, BLK), lambda i, j: (i, 0, j)),
            pl.BlockSpec((BLK, DL), lambda i, j, _nb=nb: (i * _nb + j, 0)),
            pl.BlockSpec((ND, DE), lambda i, j: (0, 0)),
            pl.BlockSpec((NT, DE), lambda i, j: (0, 0)),
            pl.BlockSpec((F + 2 * DE + DL, H), lambda i, j: (0, 0)),
            pl.BlockSpec((1, H), lambda i, j: (0, 0)),
        ],
        out_specs=pl.BlockSpec((1, BLK, H), lambda i, j: (i, j, 0)),
        out_shape=jax.ShapeDtypeStruct((T, B, H), jnp.float32),
    )


def kernel(data_num, data_cat, emb_day, emb_time, emb_loc, W_in, b_in):
    B, T, F = data_num.shape
    N = B * T
    V, DL = emb_loc.shape
    NT, DE = emb_time.shape
    H = W_in.shape[1]

    dnT = jnp.transpose(data_num, (1, 2, 0))
    catT = jnp.transpose(data_cat, (1, 2, 0))
    loc_idx = catT[:, 2, :].reshape(N // 128, 128)

    loc_rows = _make_sc_gather(V, DL, N)(emb_loc, loc_idx)

    ND = 8
    ed = jnp.zeros((ND, DE), emb_day.dtype).at[:emb_day.shape[0]].set(emb_day)

    out3 = _make_tc_fused(B, T, F, NT, DE, DL, H, BLK=4096)(
        dnT, catT, loc_rows, ed, emb_time, W_in, b_in.reshape(1, H))
    return jnp.transpose(out3, (1, 0, 2))
